# Initial kernel scaffold; baseline (speedup 1.0000x reference)
import jax
import jax.numpy as jnp
from jax.experimental import pallas as pl


def kernel(x, edge_index, W_self1, W_neigh1, b1, W_self2, W_neigh2, b2, W_score, b_score):
    raise NotImplementedError("write your pallas kernel here")


# speedup vs baseline: 7.1744x; 7.1744x over previous
import functools

import jax
import jax.numpy as jnp
from jax import lax
from jax.experimental import pallas as pl
from jax.experimental.pallas import tpu as pltpu
from jax.experimental.pallas import tpu_sc as plsc

N = 10000
E = 320000
D = 128
NCLS = 64

NC = 2
NS = 16
CH = 80
EDGES_PER_TILE = E // (NC * NS)
NCHUNK = EDGES_PER_TILE // CH
RCH = 80
NRCHUNK = N // RCH
DEGW = 16


def _fill_rows(ref, nrows, ncols, value):
    vec = jnp.full((16,), value, jnp.float32)

    def body(i, carry):
        for j in range(ncols // 16):
            ref[i, pl.ds(j * 16, 16)] = vec
        return carry

    lax.fori_loop(0, nrows, body, 0)


def _sc_segment_sum(h, edge_index, with_deg):
    mesh = plsc.VectorSubcoreMesh(
        core_axis_name="c", subcore_axis_name="s", num_cores=NC, num_subcores=NS
    )
    out_type = [jax.ShapeDtypeStruct((NC, N, D), jnp.float32)]
    scratch = [
        pltpu.VMEM_SHARED((N, D), jnp.float32),
        pltpu.VMEM((CH,), jnp.int32),
        pltpu.VMEM((CH,), jnp.int32),
        pltpu.VMEM((CH, D), jnp.float32),
        pltpu.SemaphoreType.DMA,
    ]
    if with_deg:
        out_type.append(jax.ShapeDtypeStruct((NC, N, DEGW), jnp.float32))
        scratch += [
            pltpu.VMEM_SHARED((N, DEGW), jnp.float32),
            pltpu.VMEM((CH, DEGW), jnp.float32),
            pltpu.VMEM((CH, DEGW), jnp.float32),
        ]

    def body(h_hbm, src_hbm, dst_hbm, *refs):
        if with_deg:
            (acc_out, deg_out, acc, src_v, dst_v, rows_v, sem,
             deg, ones_v, dzero_v) = refs
        else:
            (acc_out, acc, src_v, dst_v, rows_v, sem) = refs
        c = lax.axis_index("c")
        s = lax.axis_index("s")

        _fill_rows(rows_v, RCH, D, 0.0)
        if with_deg:
            _fill_rows(dzero_v, RCH, DEGW, 0.0)
            _fill_rows(ones_v, CH, DEGW, 1.0)

        def zero_chunk(i, carry):
            m = s + i * NS

            @pl.when(m < NRCHUNK)
            def _():
                pltpu.sync_copy(rows_v, acc.at[pl.ds(m * RCH, RCH)])
                if with_deg:
                    pltpu.sync_copy(dzero_v, deg.at[pl.ds(m * RCH, RCH)])
            return carry

        lax.fori_loop(0, (NRCHUNK + NS - 1) // NS, zero_chunk, 0)
        plsc.subcore_barrier()

        tile_base = c * (E // NC) + s * EDGES_PER_TILE

        def chunk(k, carry):
            base = tile_base + k * CH
            pltpu.sync_copy(src_hbm.at[pl.ds(base, CH)], src_v)
            pltpu.sync_copy(dst_hbm.at[pl.ds(base, CH)], dst_v)
            pltpu.async_copy(h_hbm.at[src_v], rows_v, sem).wait()
            pltpu.sync_copy(rows_v, acc.at[dst_v], add=True)
            if with_deg:
                pltpu.sync_copy(ones_v, deg.at[dst_v], add=True)
            return carry

        lax.fori_loop(0, NCHUNK, chunk, 0)
        plsc.subcore_barrier()

        def out_chunk(i, carry):
            m = s + i * NS

            @pl.when(m < NRCHUNK)
            def _():
                pltpu.sync_copy(acc.at[pl.ds(m * RCH, RCH)],
                                acc_out.at[c, pl.ds(m * RCH, RCH)])
                if with_deg:
                    pltpu.sync_copy(deg.at[pl.ds(m * RCH, RCH)],
                                    deg_out.at[c, pl.ds(m * RCH, RCH)])
            return carry

        lax.fori_loop(0, (NRCHUNK + NS - 1) // NS, out_chunk, 0)

    f = pl.kernel(body, out_type=out_type, mesh=mesh,
                  name="sc_segment_sum" + ("_deg" if with_deg else ""),
                  compiler_params=pltpu.CompilerParams(use_tc_tiling_on_sc=False),
                  scratch_types=scratch)
    return f(h, edge_index[0], edge_index[1])


_TC_R = 1000


def _tc1_body(x_ref, acc_ref, deg_ref, ws_ref, wn_ref, b_ref, out_ref):
    x = x_ref[...]
    agg = acc_ref[0] + acc_ref[1] + x
    deg = (jnp.sum(deg_ref[0], axis=1, keepdims=True)
           + jnp.sum(deg_ref[1], axis=1, keepdims=True)) * (1.0 / DEGW) + 1.0
    mean = agg / deg
    h = (jnp.dot(x, ws_ref[...], preferred_element_type=jnp.float32)
         + jnp.dot(mean, wn_ref[...], preferred_element_type=jnp.float32)
         + b_ref[...])
    out_ref[...] = jnp.where(h >= 0.0, h, 0.01 * h)


def _tc2_body(h_ref, acc_ref, deg_ref, ws_ref, wn_ref, b_ref, wsc_ref,
              bsc_ref, out_ref):
    h = h_ref[...]
    agg = acc_ref[0] + acc_ref[1] + h
    deg = (jnp.sum(deg_ref[0], axis=1, keepdims=True)
           + jnp.sum(deg_ref[1], axis=1, keepdims=True)) * (1.0 / DEGW) + 1.0
    mean = agg / deg
    h2 = (jnp.dot(h, ws_ref[...], preferred_element_type=jnp.float32)
          + jnp.dot(mean, wn_ref[...], preferred_element_type=jnp.float32)
          + b_ref[...])
    out_ref[...] = (jnp.dot(h2, wsc_ref[...], preferred_element_type=jnp.float32)
                    + bsc_ref[...])


def _row_spec(r, d):
    return pl.BlockSpec((r, d), lambda i: (i, 0))


def _stack_spec(r, d):
    return pl.BlockSpec((NC, r, d), lambda i: (0, i, 0))


def _full_spec(a, b):
    return pl.BlockSpec((a, b), lambda i: (0, 0))


def _tc_layer1(x, acc, deg, ws, wn, b):
    return pl.pallas_call(
        _tc1_body,
        grid=(N // _TC_R,),
        in_specs=[
            _row_spec(_TC_R, D), _stack_spec(_TC_R, D), _stack_spec(_TC_R, DEGW),
            _full_spec(D, D), _full_spec(D, D), _full_spec(1, D),
        ],
        out_specs=_row_spec(_TC_R, D),
        out_shape=jax.ShapeDtypeStruct((N, D), jnp.float32),
    )(x, acc, deg, ws, wn, b.reshape(1, D))


def _tc_layer2(h, acc, deg, ws, wn, b, wsc, bsc):
    return pl.pallas_call(
        _tc2_body,
        grid=(N // _TC_R,),
        in_specs=[
            _row_spec(_TC_R, D), _stack_spec(_TC_R, D), _stack_spec(_TC_R, DEGW),
            _full_spec(D, D), _full_spec(D, D), _full_spec(1, D),
            _full_spec(D, NCLS), _full_spec(1, NCLS),
        ],
        out_specs=_row_spec(_TC_R, NCLS),
        out_shape=jax.ShapeDtypeStruct((N, NCLS), jnp.float32),
    )(h, acc, deg, ws, wn, b.reshape(1, D), wsc, bsc.reshape(1, NCLS))


def kernel(x, edge_index, W_self1, W_neigh1, b1, W_self2, W_neigh2, b2,
           W_score, b_score):
    acc1, deg = _sc_segment_sum(x, edge_index, with_deg=True)
    h1 = _tc_layer1(x, acc1, deg, W_self1, W_neigh1, b1)
    (acc2,) = _sc_segment_sum(h1, edge_index, with_deg=False)
    return _tc_layer2(h1, acc2, deg, W_self2, W_neigh2, b2, W_score, b_score)


# speedup vs baseline: 13.1466x; 1.8324x over previous
import functools

import jax
import jax.numpy as jnp
from jax import lax
from jax.experimental import pallas as pl
from jax.experimental.pallas import tpu as pltpu
from jax.experimental.pallas import tpu_sc as plsc

N = 10000
E = 320000
D = 128
NCLS = 64

NC = 2
NS = 16
CH = 80
EDGES_PER_TILE = E // (NC * NS)
NCHUNK = EDGES_PER_TILE // CH
RCH = 80
NRCHUNK = N // RCH
DEGW = 16


def _fill_rows(ref, nrows, ncols, value):
    vec = jnp.full((16,), value, jnp.float32)

    def body(i, carry):
        for j in range(ncols // 16):
            ref[i, pl.ds(j * 16, 16)] = vec
        return carry

    lax.fori_loop(0, nrows, body, 0)


def _sc_segment_sum(h, edge_index, with_deg):
    mesh = plsc.VectorSubcoreMesh(
        core_axis_name="c", subcore_axis_name="s", num_cores=NC, num_subcores=NS
    )
    out_type = [jax.ShapeDtypeStruct((NC, N, D), jnp.float32)]
    scratch = [
        pltpu.VMEM_SHARED((N, D), jnp.float32),
        pltpu.VMEM((2, CH), jnp.int32),
        pltpu.VMEM((2, CH), jnp.int32),
        pltpu.VMEM((2, CH, D), jnp.float32),
        pltpu.VMEM((RCH, D), jnp.float32),
        pltpu.SemaphoreType.DMA,
        pltpu.SemaphoreType.DMA,
    ]
    if with_deg:
        out_type.append(jax.ShapeDtypeStruct((NC, N, DEGW), jnp.float32))
        scratch += [
            pltpu.VMEM_SHARED((N, DEGW), jnp.float32),
            pltpu.VMEM((CH, DEGW), jnp.float32),
            pltpu.VMEM((RCH, DEGW), jnp.float32),
        ]

    def body(h_hbm, src_hbm, dst_hbm, *refs):
        if with_deg:
            (acc_out, deg_out, acc, src_i, dst_i, rows2, zrows_v, gsem, isem,
             deg, ones_v, dzero_v) = refs
        else:
            (acc_out, acc, src_i, dst_i, rows2, zrows_v, gsem, isem) = refs
        c = lax.axis_index("c")
        s = lax.axis_index("s")

        _fill_rows(zrows_v, RCH, D, 0.0)
        if with_deg:
            _fill_rows(dzero_v, RCH, DEGW, 0.0)
            _fill_rows(ones_v, CH, DEGW, 1.0)

        def zero_chunk(i, carry):
            m = s + i * NS

            @pl.when(m < NRCHUNK)
            def _():
                pltpu.sync_copy(zrows_v, acc.at[pl.ds(m * RCH, RCH)])
                if with_deg:
                    pltpu.sync_copy(dzero_v, deg.at[pl.ds(m * RCH, RCH)])
            return carry

        lax.fori_loop(0, (NRCHUNK + NS - 1) // NS, zero_chunk, 0)
        plsc.subcore_barrier()

        tile_base = c * (E // NC) + s * EDGES_PER_TILE

        def _idx_wait(slot):
            pltpu.make_async_copy(src_hbm.at[pl.ds(0, CH)],
                                  src_i.at[slot], isem).wait()
            pltpu.make_async_copy(dst_hbm.at[pl.ds(0, CH)],
                                  dst_i.at[slot], isem).wait()

        pltpu.sync_copy(src_hbm.at[pl.ds(tile_base, CH)], src_i.at[0])
        pltpu.sync_copy(dst_hbm.at[pl.ds(tile_base, CH)], dst_i.at[0])
        pltpu.async_copy(h_hbm.at[src_i.at[0]], rows2.at[0], gsem)
        pltpu.async_copy(src_hbm.at[pl.ds(tile_base + CH, CH)],
                         src_i.at[1], isem)
        pltpu.async_copy(dst_hbm.at[pl.ds(tile_base + CH, CH)],
                         dst_i.at[1], isem)

        def it(k, carry):
            cur = lax.rem(k, 2)
            nxt = 1 - cur

            @pl.when(k < NCHUNK - 1)
            def _():
                _idx_wait(nxt)

            pltpu.make_async_copy(h_hbm.at[src_i.at[cur]],
                                  rows2.at[cur], gsem).wait()

            @pl.when(k < NCHUNK - 1)
            def _():
                pltpu.async_copy(h_hbm.at[src_i.at[nxt]], rows2.at[nxt], gsem)

            pltpu.sync_copy(rows2.at[cur], acc.at[dst_i.at[cur]], add=True)
            if with_deg:
                pltpu.sync_copy(ones_v, deg.at[dst_i.at[cur]], add=True)

            @pl.when(k < NCHUNK - 2)
            def _():
                base = tile_base + (k + 2) * CH
                pltpu.async_copy(src_hbm.at[pl.ds(base, CH)],
                                 src_i.at[cur], isem)
                pltpu.async_copy(dst_hbm.at[pl.ds(base, CH)],
                                 dst_i.at[cur], isem)
            return carry

        lax.fori_loop(0, NCHUNK, it, 0)
        plsc.subcore_barrier()

        def out_chunk(i, carry):
            m = s + i * NS

            @pl.when(m < NRCHUNK)
            def _():
                pltpu.sync_copy(acc.at[pl.ds(m * RCH, RCH)],
                                acc_out.at[c, pl.ds(m * RCH, RCH)])
                if with_deg:
                    pltpu.sync_copy(deg.at[pl.ds(m * RCH, RCH)],
                                    deg_out.at[c, pl.ds(m * RCH, RCH)])
            return carry

        lax.fori_loop(0, (NRCHUNK + NS - 1) // NS, out_chunk, 0)

    f = pl.kernel(body, out_type=out_type, mesh=mesh,
                  name="sc_segment_sum" + ("_deg" if with_deg else ""),
                  compiler_params=pltpu.CompilerParams(use_tc_tiling_on_sc=False),
                  scratch_types=scratch)
    return f(h, edge_index[0], edge_index[1])


_TC_R = 1000


def _tc1_body(x_ref, acc_ref, deg_ref, ws_ref, wn_ref, b_ref, out_ref):
    x = x_ref[...]
    agg = acc_ref[0] + acc_ref[1] + x
    deg = (jnp.sum(deg_ref[0], axis=1, keepdims=True)
           + jnp.sum(deg_ref[1], axis=1, keepdims=True)) * (1.0 / DEGW) + 1.0
    mean = agg / deg
    h = (jnp.dot(x, ws_ref[...], preferred_element_type=jnp.float32)
         + jnp.dot(mean, wn_ref[...], preferred_element_type=jnp.float32)
         + b_ref[...])
    out_ref[...] = jnp.where(h >= 0.0, h, 0.01 * h)


def _tc2_body(h_ref, acc_ref, deg_ref, ws_ref, wn_ref, b_ref, wsc_ref,
              bsc_ref, out_ref):
    h = h_ref[...]
    agg = acc_ref[0] + acc_ref[1] + h
    deg = (jnp.sum(deg_ref[0], axis=1, keepdims=True)
           + jnp.sum(deg_ref[1], axis=1, keepdims=True)) * (1.0 / DEGW) + 1.0
    mean = agg / deg
    h2 = (jnp.dot(h, ws_ref[...], preferred_element_type=jnp.float32)
          + jnp.dot(mean, wn_ref[...], preferred_element_type=jnp.float32)
          + b_ref[...])
    out_ref[...] = (jnp.dot(h2, wsc_ref[...], preferred_element_type=jnp.float32)
                    + bsc_ref[...])


def _row_spec(r, d):
    return pl.BlockSpec((r, d), lambda i: (i, 0))


def _stack_spec(r, d):
    return pl.BlockSpec((NC, r, d), lambda i: (0, i, 0))


def _full_spec(a, b):
    return pl.BlockSpec((a, b), lambda i: (0, 0))


def _tc_layer1(x, acc, deg, ws, wn, b):
    return pl.pallas_call(
        _tc1_body,
        grid=(N // _TC_R,),
        in_specs=[
            _row_spec(_TC_R, D), _stack_spec(_TC_R, D), _stack_spec(_TC_R, DEGW),
            _full_spec(D, D), _full_spec(D, D), _full_spec(1, D),
        ],
        out_specs=_row_spec(_TC_R, D),
        out_shape=jax.ShapeDtypeStruct((N, D), jnp.float32),
    )(x, acc, deg, ws, wn, b.reshape(1, D))


def _tc_layer2(h, acc, deg, ws, wn, b, wsc, bsc):
    return pl.pallas_call(
        _tc2_body,
        grid=(N // _TC_R,),
        in_specs=[
            _row_spec(_TC_R, D), _stack_spec(_TC_R, D), _stack_spec(_TC_R, DEGW),
            _full_spec(D, D), _full_spec(D, D), _full_spec(1, D),
            _full_spec(D, NCLS), _full_spec(1, NCLS),
        ],
        out_specs=_row_spec(_TC_R, NCLS),
        out_shape=jax.ShapeDtypeStruct((N, NCLS), jnp.float32),
    )(h, acc, deg, ws, wn, b.reshape(1, D), wsc, bsc.reshape(1, NCLS))


def kernel(x, edge_index, W_self1, W_neigh1, b1, W_self2, W_neigh2, b2,
           W_score, b_score):
    acc1, deg = _sc_segment_sum(x, edge_index, with_deg=True)
    h1 = _tc_layer1(x, acc1, deg, W_self1, W_neigh1, b1)
    (acc2,) = _sc_segment_sum(h1, edge_index, with_deg=False)
    return _tc_layer2(h1, acc2, deg, W_self2, W_neigh2, b2, W_score, b_score)
